# baseline (speedup 1.0000x reference)
import jax
import jax.numpy as jnp
from jax.experimental import pallas as pl
from jax.experimental.pallas import tpu as pltpu

_DIM = 64
_K = 512
_BB = 32
_BBC = 32


def _stride2_cell_weights(w):
    o, c = w.shape[0], w.shape[1]
    zero = jnp.zeros((c, o), jnp.float32)
    rows = []
    for cy in (-1, 0, 1):
        cols = []
        for cx in (-1, 0, 1):
            blocks = []
            for py in (0, 1):
                for px in (0, 1):
                    ky, kx = 2 * cy + py + 1, 2 * cx + px + 1
                    if 0 <= ky < 4 and 0 <= kx < 4:
                        blocks.append(w[:, :, ky, kx].T)
                    else:
                        blocks.append(zero)
            cols.append(jnp.concatenate(blocks, axis=0))
        rows.append(jnp.stack(cols))
    return jnp.stack(rows)


def _convT_cell_weights(w):
    o, c = w.shape[0], w.shape[1]
    zero = jnp.zeros((c, o), jnp.float32)
    rows = []
    for cy in (-1, 0, 1):
        cols = []
        for cx in (-1, 0, 1):
            blocks = []
            for a in (0, 1):
                for b in (0, 1):
                    ky, kx = 2 - a + 2 * cy, 2 - b + 2 * cx
                    if 0 <= ky < 4 and 0 <= kx < 4:
                        blocks.append(w[:, :, ky, kx].T)
                    else:
                        blocks.append(zero)
            cols.append(jnp.concatenate(blocks, axis=1))
        rows.append(jnp.stack(cols))
    return jnp.stack(rows)


def _plain_cell_weights(w):
    return jnp.transpose(w, (2, 3, 1, 0))


def _fold_bn(wc, b, g, be):
    return wc * g[None, None, None, :], (b * g + be)[None, :]


def _cellconv3x3(h, w_taps, bias, scratch):
    bb, hh, ww, c = h.shape
    n = w_taps.shape[-1]
    scratch[:, 1:hh + 1, 1:ww + 1, :] = h
    m = bb * hh * ww
    acc = jnp.broadcast_to(bias, (m, n))
    for cy in range(3):
        for cx in range(3):
            v = scratch[:, cy:cy + hh, cx:cx + ww, :].reshape(m, c)
            acc = acc + jnp.dot(v, w_taps[cy, cx],
                                preferred_element_type=jnp.float32)
    return acc.reshape(bb, hh, ww, n)


def _k_cellconv(x_ref, w_ref, b_ref, o_ref, sc, *, relu):
    @pl.when(pl.program_id(0) == 0)
    def _init():
        sc[...] = jnp.zeros_like(sc)

    h = _cellconv3x3(x_ref[...], w_ref[...], b_ref[...], sc)
    o_ref[...] = jnp.maximum(h, 0.0) if relu else h


def _run_cellconv(xc, wc, b, relu):
    bb = _BB
    bsz, hh, _, c = xc.shape
    n = wc.shape[-1]
    import functools
    return pl.pallas_call(
        functools.partial(_k_cellconv, relu=relu),
        grid=(bsz // bb,),
        in_specs=[
            pl.BlockSpec((bb, hh, hh, c), lambda i: (i, 0, 0, 0)),
            pl.BlockSpec((3, 3, c, n), lambda i: (0, 0, 0, 0)),
            pl.BlockSpec((1, n), lambda i: (0, 0)),
        ],
        out_specs=pl.BlockSpec((bb, hh, hh, n), lambda i: (i, 0, 0, 0)),
        out_shape=jax.ShapeDtypeStruct((bsz, hh, hh, n), jnp.float32),
        scratch_shapes=[pltpu.VMEM((bb, hh + 2, hh + 2, c), jnp.float32)],
        compiler_params=pltpu.CompilerParams(
            dimension_semantics=("arbitrary",)),
    )(xc, wc, b)


def _k_body(h_ref, w3_ref, b3_ref, rbw1_ref, rbb1_ref, rbw2_ref, rbb2_ref,
            cb_ref, cbt_ref, wd1_ref, bd1_ref,
            o_ref, loss_ref, perp_ref,
            sc, sq_acc, cnt_acc):
    step = pl.program_id(0)
    nsteps = pl.num_programs(0)

    @pl.when(step == 0)
    def _init():
        sc[...] = jnp.zeros_like(sc)
        sq_acc[...] = jnp.zeros_like(sq_acc)
        cnt_acc[...] = jnp.zeros_like(cnt_acc)

    bb = h_ref.shape[0]
    m = bb * 64

    def resblock(h, j):
        t = jnp.maximum(h, 0.0)
        t = _cellconv3x3(t, rbw1_ref[j], rbb1_ref[j:j + 1, :], sc)
        t = jnp.maximum(t, 0.0)
        t2 = jnp.dot(t.reshape(m, _DIM), rbw2_ref[j],
                     preferred_element_type=jnp.float32) + rbb2_ref[j:j + 1, :]
        return h + t2.reshape(bb, 8, 8, _DIM)

    h = _cellconv3x3(h_ref[...], w3_ref[...], b3_ref[...], sc)
    h = resblock(h, 0)
    h = resblock(h, 1)

    ze = h.reshape(m, _DIM)
    cbt = cbt_ref[...]
    csq = jnp.sum(cbt * cbt, axis=0, keepdims=True)
    zsq = jnp.sum(ze * ze, axis=1, keepdims=True)
    d = zsq - 2.0 * jnp.dot(ze, cbt, preferred_element_type=jnp.float32) + csq
    dmin = jnp.min(d, axis=1, keepdims=True)
    iota = jax.lax.broadcasted_iota(jnp.int32, (m, _K), 1)
    idx = jnp.min(jnp.where(d == dmin, iota, _K), axis=1, keepdims=True)
    onehot = (iota == idx).astype(jnp.float32)
    zq = jnp.dot(onehot, cb_ref[...], preferred_element_type=jnp.float32)

    diff = ze - zq
    sq_acc[...] = sq_acc[...] + jnp.sum(diff * diff, keepdims=True)
    cnt_acc[...] = cnt_acc[...] + jnp.sum(onehot, axis=0, keepdims=True)

    hq = zq.reshape(bb, 8, 8, _DIM)
    hq = resblock(hq, 2)
    hq = resblock(hq, 3)
    hd = jnp.maximum(hq, 0.0)
    hd = _cellconv3x3(hd, wd1_ref[...], bd1_ref[...], sc)
    o_ref[...] = jnp.maximum(hd, 0.0)

    @pl.when(step == nsteps - 1)
    def _fin():
        total = jnp.float32(nsteps * m)
        loss_ref[...] = 1.25 * sq_acc[...] / (total * _DIM)
        probs = cnt_acc[...] / total
        ent = -jnp.sum(probs * jnp.log(probs + 1e-10), keepdims=True)
        perp_ref[...] = jnp.exp(ent)


def _run_body(h2, w3, b3, rbw1, rbb1, rbw2, rbb2, cb, cbt, wd1, bd1):
    bb = _BBC
    bsz = h2.shape[0]
    return pl.pallas_call(
        _k_body,
        grid=(bsz // bb,),
        in_specs=[
            pl.BlockSpec((bb, 8, 8, _DIM), lambda i: (i, 0, 0, 0)),
            pl.BlockSpec((3, 3, _DIM, _DIM), lambda i: (0, 0, 0, 0)),
            pl.BlockSpec((1, _DIM), lambda i: (0, 0)),
            pl.BlockSpec((4, 3, 3, _DIM, _DIM), lambda i: (0, 0, 0, 0, 0)),
            pl.BlockSpec((4, _DIM), lambda i: (0, 0)),
            pl.BlockSpec((4, _DIM, _DIM), lambda i: (0, 0, 0)),
            pl.BlockSpec((4, _DIM), lambda i: (0, 0)),
            pl.BlockSpec((_K, _DIM), lambda i: (0, 0)),
            pl.BlockSpec((_DIM, _K), lambda i: (0, 0)),
            pl.BlockSpec((3, 3, _DIM, 2 * _DIM), lambda i: (0, 0, 0, 0)),
            pl.BlockSpec((1, 2 * _DIM), lambda i: (0, 0)),
        ],
        out_specs=[
            pl.BlockSpec((bb, 8, 8, 2 * _DIM), lambda i: (i, 0, 0, 0)),
            pl.BlockSpec((1, 1), lambda i: (0, 0)),
            pl.BlockSpec((1, 1), lambda i: (0, 0)),
        ],
        out_shape=[
            jax.ShapeDtypeStruct((bsz, 8, 8, 2 * _DIM), jnp.float32),
            jax.ShapeDtypeStruct((1, 1), jnp.float32),
            jax.ShapeDtypeStruct((1, 1), jnp.float32),
        ],
        scratch_shapes=[
            pltpu.VMEM((bb, 10, 10, _DIM), jnp.float32),
            pltpu.VMEM((1, 1), jnp.float32),
            pltpu.VMEM((1, _K), jnp.float32),
        ],
        compiler_params=pltpu.CompilerParams(
            dimension_semantics=("arbitrary",)),
    )(h2, w3, b3, rbw1, rbb1, rbw2, rbb2, cb, cbt, wd1, bd1)


def _s2d(a):
    b, h, w, c = a.shape
    a = a.reshape(b, h // 2, 2, w // 2, 2, c)
    return a.transpose(0, 1, 3, 2, 4, 5).reshape(b, h // 2, w // 2, 4 * c)


def _d2s(a):
    b, h, w, c4 = a.shape
    c = c4 // 4
    a = a.reshape(b, h, w, 2, 2, c)
    return a.transpose(0, 1, 3, 2, 4, 5).reshape(b, 2 * h, 2 * w, c)


@jax.jit
def _vq_vae(x, params):
    p = params
    wc1, bc1 = _fold_bn(_stride2_cell_weights(p['e_c1_w']), p['e_c1_b'],
                        p['e_bn1_g'], p['e_bn1_b'])
    wc2, bc2 = _fold_bn(_stride2_cell_weights(p['e_c2_w']), p['e_c2_b'],
                        p['e_bn2_g'], p['e_bn2_b'])
    w3 = _plain_cell_weights(p['e_c3_w'])
    b3 = p['e_c3_b'][None, :]
    rbw1, rbb1, rbw2, rbb2 = [], [], [], []
    for pref in ('e_rb0', 'e_rb1', 'd_rb0', 'd_rb1'):
        w1, bb1 = _fold_bn(_plain_cell_weights(p[pref + '_w1']),
                           p[pref + '_b1'], p[pref + '_g1'], p[pref + '_be1'])
        rbw1.append(w1)
        rbb1.append(bb1[0])
        rbw2.append(p[pref + '_w2'][:, :, 0, 0].T * p[pref + '_g2'][None, :])
        rbb2.append(p[pref + '_b2'] * p[pref + '_g2'] + p[pref + '_be2'])
    rbw1 = jnp.stack(rbw1)
    rbb1 = jnp.stack(rbb1)
    rbw2 = jnp.stack(rbw2)
    rbb2 = jnp.stack(rbb2)
    wd1, bd1 = _fold_bn(_convT_cell_weights(p['d_c1_w']),
                        jnp.tile(p['d_c1_b'], 4), jnp.tile(p['d_bn1_g'], 4),
                        jnp.tile(p['d_bn1_b'], 4))
    wd2 = _convT_cell_weights(p['d_c2_w'])
    bd2 = jnp.tile(p['d_c2_b'], 4)[None, :]
    cb = p['codebook']
    cbt = cb.T

    xc = _s2d(jnp.transpose(x, (0, 2, 3, 1)))
    h1 = _run_cellconv(xc, wc1, bc1, True)
    h1c = _s2d(h1)
    h2 = _run_cellconv(h1c, wc2, bc2, True)
    hd, loss, perp = _run_body(h2, w3, b3, rbw1, rbb1, rbw2, rbb2,
                               cb, cbt, wd1, bd1)
    hs = _d2s(hd)
    out = _run_cellconv(hs, wd2, bd2, False)
    recon = jnp.transpose(_d2s(out), (0, 3, 1, 2))
    return recon, loss[0, 0], perp[0, 0]


def kernel(x, params):
    return _vq_vae(x, params)


# speedup vs baseline: 2.2027x; 2.2027x over previous
import jax
import jax.numpy as jnp
import numpy as np
from jax.experimental import pallas as pl
from jax.experimental.pallas import tpu as pltpu

_DIM = 64
_K = 512
_BB = 32


def _np_supercell_conv1(w):
    o, c = w.shape[0], w.shape[1]
    out = np.zeros((3, 3, 48, 128), np.float32)
    for cy in (-1, 0, 1):
        for cx in (-1, 0, 1):
            for u in range(4):
                for v in range(4):
                    for py in (0, 1):
                        for px in (0, 1):
                            ky = 4 * cy + u + 1 - 2 * py
                            kx = 4 * cx + v + 1 - 2 * px
                            if 0 <= ky < 4 and 0 <= kx < 4:
                                r0, c0 = u * 12 + v * 3, py * 64 + px * 32
                                out[cy + 1, cx + 1, r0:r0 + 3,
                                    c0:c0 + 32] = w[:, :, ky, kx].T
    return out


def _np_stride2_cell(w):
    o, c = w.shape[0], w.shape[1]
    out = np.zeros((3, 3, 4 * c, o), np.float32)
    for cy in (-1, 0, 1):
        for cx in (-1, 0, 1):
            for py in (0, 1):
                for px in (0, 1):
                    ky, kx = 2 * cy + py + 1, 2 * cx + px + 1
                    if 0 <= ky < 4 and 0 <= kx < 4:
                        s = py * 2 * c + px * c
                        out[cy + 1, cx + 1, s:s + c, :] = w[:, :, ky, kx].T
    return out


def _np_convT_cell(w):
    o, c = w.shape[0], w.shape[1]
    out = np.zeros((3, 3, c, 4 * o), np.float32)
    for cy in (-1, 0, 1):
        for cx in (-1, 0, 1):
            for a in (0, 1):
                for b in (0, 1):
                    ky, kx = 2 - a + 2 * cy, 2 - b + 2 * cx
                    if 0 <= ky < 4 and 0 <= kx < 4:
                        s = a * 2 * o + b * o
                        out[cy + 1, cx + 1, :, s:s + o] = w[:, :, ky, kx].T
    return out


def _np_supercell_convT2(w):
    out = np.zeros((3, 3, 128, 48), np.float32)
    for cy in (-1, 0, 1):
        for cx in (-1, 0, 1):
            for pin in (0, 1):
                for pjn in (0, 1):
                    for u in range(4):
                        for v in range(4):
                            pyp, a = u // 2, u % 2
                            pxp, b = v // 2, v % 2
                            cy2 = 2 * cy + pin - pyp
                            cx2 = 2 * cx + pjn - pxp
                            ky, kx = 2 - a + 2 * cy2, 2 - b + 2 * cx2
                            if 0 <= ky < 4 and 0 <= kx < 4:
                                r0, c0 = pin * 64 + pjn * 32, u * 12 + v * 3
                                out[cy + 1, cx + 1, r0:r0 + 32,
                                    c0:c0 + 3] = w[:, :, ky, kx].T
    return out


def _extract_places(builder, oc, rblk, cblk):
    o, c = oc
    codes = np.arange(1, 17, dtype=np.float32).reshape(4, 4)
    w = np.broadcast_to(codes, (o, c, 4, 4)).copy()
    out = builder(w)
    _, _, rr, nn = out.shape
    places = []
    for i in range(3):
        for j in range(3):
            for r0 in range(0, rr, rblk):
                for c0 in range(0, nn, cblk):
                    v = out[i, j, r0, c0]
                    if v != 0:
                        blk = out[i, j, r0:r0 + rblk, c0:c0 + cblk]
                        assert np.all(blk == v)
                        t = int(v) - 1
                        places.append(((i * 3 + j) * rr + r0, c0,
                                       t // 4, t % 4))
    return places


_PLACE_SC1 = _extract_places(_np_supercell_conv1, (32, 3), 3, 32)
_PLACE_SC2 = _extract_places(_np_stride2_cell, (64, 32), 32, 64)
_PLACE_D1 = _extract_places(_np_convT_cell, (32, 64), 64, 32)
_PLACE_D2 = _extract_places(_np_supercell_convT2, (3, 32), 32, 3)


def _cellconv(h, w, s, t, scratch):
    bb, hh, ww, c = h.shape
    scratch[:, 1:hh + 1, 1:ww + 1, :] = h
    m = bb * hh * ww
    cols = [scratch[:, cy:cy + hh, cx:cx + ww, :].reshape(m, c)
            for cy in range(3) for cx in range(3)]
    v = jnp.concatenate(cols, axis=1)
    out = jnp.dot(v, w, preferred_element_type=jnp.float32)
    out = out * s + t if s is not None else out + t
    return out.reshape(bb, hh, ww, w.shape[-1])


def _k_net(xs_ref, w1t_ref, w2t_ref, w33_ref, wrb2_ref, cb_ref,
           d1t_ref, d2t_ref,
           b1_ref, g1_ref, be1_ref, b2_ref, g2_ref, be2_ref, b3_ref,
           rb_b1_ref, rb_g1_ref, rb_be1_ref, rb_b2_ref, rb_g2_ref,
           rb_be2_ref, bd1_ref, gd1_ref, bed1_ref, bd2_ref,
           o_ref, loss_ref, perp_ref,
           sc48, sc128, sc64, wsc1_s, wsc2_s, wd1_s, wsct2_s,
           sq_acc, cnt_acc):
    step = pl.program_id(0)
    nsteps = pl.num_programs(0)

    @pl.when(step == 0)
    def _init():
        sc48[...] = jnp.zeros_like(sc48)
        sc128[...] = jnp.zeros_like(sc128)
        sc64[...] = jnp.zeros_like(sc64)
        sq_acc[...] = jnp.zeros_like(sq_acc)
        cnt_acc[...] = jnp.zeros_like(cnt_acc)
        wsc1_s[...] = jnp.zeros_like(wsc1_s)
        wsc2_s[...] = jnp.zeros_like(wsc2_s)
        wd1_s[...] = jnp.zeros_like(wd1_s)
        wsct2_s[...] = jnp.zeros_like(wsct2_s)
        for r0, c0, ky, kx in _PLACE_SC1:
            wsc1_s[r0:r0 + 3, c0:c0 + 32] = w1t_ref[ky, kx]
        for r0, c0, ky, kx in _PLACE_SC2:
            wsc2_s[r0:r0 + 32, c0:c0 + 64] = w2t_ref[ky, kx]
        for r0, c0, ky, kx in _PLACE_D1:
            wd1_s[r0:r0 + 64, c0:c0 + 32] = d1t_ref[ky, kx]
        for r0, c0, ky, kx in _PLACE_D2:
            wsct2_s[r0:r0 + 32, c0:c0 + 3] = d2t_ref[ky, kx]

    bb = xs_ref.shape[0]
    m = bb * 64

    def tile4(row):
        return jnp.concatenate([row, row, row, row], axis=1)

    def resblock(h, j):
        t = jnp.maximum(h, 0.0)
        g = rb_g1_ref[j:j + 1, :]
        tb = rb_b1_ref[j:j + 1, :] * g + rb_be1_ref[j:j + 1, :]
        t = _cellconv(t, w33_ref[j + 1], g, tb, sc64)
        t = jnp.maximum(t, 0.0)
        g2 = rb_g2_ref[j:j + 1, :]
        tb2 = rb_b2_ref[j:j + 1, :] * g2 + rb_be2_ref[j:j + 1, :]
        t2 = jnp.dot(t.reshape(m, _DIM), wrb2_ref[j],
                     preferred_element_type=jnp.float32) * g2 + tb2
        return h + t2.reshape(bb, 8, 8, _DIM)

    g1 = tile4(g1_ref[...])
    t1 = tile4(b1_ref[...] * g1_ref[...] + be1_ref[...])
    h = jnp.maximum(_cellconv(xs_ref[...], wsc1_s[...], g1, t1, sc48),
                    0.0)
    t2r = b2_ref[...] * g2_ref[...] + be2_ref[...]
    h = jnp.maximum(_cellconv(h, wsc2_s[...], g2_ref[...], t2r, sc128),
                    0.0)
    h = _cellconv(h, w33_ref[0], None, b3_ref[...], sc64)
    h = resblock(h, 0)
    h = resblock(h, 1)

    ze = h.reshape(m, _DIM)
    cb = cb_ref[...]
    dn = (((1,), (1,)), ((), ()))
    csq = jax.lax.dot_general(jnp.ones((1, _DIM), jnp.float32), cb * cb,
                              dn, preferred_element_type=jnp.float32)
    zsq = jnp.sum(ze * ze, axis=1, keepdims=True)
    d = zsq - 2.0 * jax.lax.dot_general(
        ze, cb, dn, preferred_element_type=jnp.float32) + csq
    dmin = jnp.min(d, axis=1, keepdims=True)
    iota = jax.lax.broadcasted_iota(jnp.int32, (m, _K), 1)
    idx = jnp.min(jnp.where(d == dmin, iota, _K), axis=1, keepdims=True)
    onehot = (iota == idx).astype(jnp.float32)
    zq = jnp.dot(onehot, cb, preferred_element_type=jnp.float32)

    diff = ze - zq
    sq_acc[...] = sq_acc[...] + jnp.sum(diff * diff, keepdims=True)
    cnt_acc[...] = cnt_acc[...] + jnp.sum(onehot, axis=0, keepdims=True)

    h = zq.reshape(bb, 8, 8, _DIM)
    h = resblock(h, 2)
    h = resblock(h, 3)
    h = jnp.maximum(h, 0.0)
    gdt = tile4(gd1_ref[...])
    tdt = tile4(bd1_ref[...] * gd1_ref[...] + bed1_ref[...])
    h = jnp.maximum(_cellconv(h, wd1_s[...], gdt, tdt, sc64),
                    0.0)
    td2 = jnp.concatenate([bd2_ref[...]] * 16, axis=1)
    o_ref[...] = _cellconv(h, wsct2_s[...], None, td2, sc128)

    @pl.when(step == nsteps - 1)
    def _fin():
        total = jnp.float32(nsteps * m)
        loss_ref[...] = 1.25 * sq_acc[...] / (total * _DIM)
        probs = cnt_acc[...] / total
        ent = -jnp.sum(probs * jnp.log(probs + 1e-10), keepdims=True)
        perp_ref[...] = jnp.exp(ent)


@jax.jit
def _run(x, params):
    p = params
    bsz = x.shape[0]
    bb = _BB

    w1t = jnp.transpose(p['e_c1_w'], (2, 3, 1, 0))
    w2t = jnp.transpose(p['e_c2_w'], (2, 3, 1, 0))
    d1t = jnp.transpose(p['d_c1_w'], (2, 3, 1, 0))
    d2t = jnp.transpose(p['d_c2_w'], (2, 3, 1, 0))
    w33 = jnp.transpose(
        jnp.stack([p['e_c3_w'], p['e_rb0_w1'], p['e_rb1_w1'],
                   p['d_rb0_w1'], p['d_rb1_w1']]),
        (0, 3, 4, 2, 1)).reshape(5, 9 * _DIM, _DIM)
    wrb2 = jnp.transpose(
        jnp.stack([p['e_rb0_w2'], p['e_rb1_w2'],
                   p['d_rb0_w2'], p['d_rb1_w2']])[:, :, :, 0, 0],
        (0, 2, 1))
    row = lambda a: a[None, :]
    st4 = lambda suf: jnp.stack([p['e_rb0' + suf], p['e_rb1' + suf],
                                 p['d_rb0' + suf], p['d_rb1' + suf]])

    xs = jnp.transpose(x, (0, 2, 3, 1)).reshape(bsz, 8, 4, 8, 4, 3)
    xs = xs.transpose(0, 1, 3, 2, 4, 5).reshape(bsz, 8, 8, 48)

    def const(*s):
        return pl.BlockSpec(s, lambda i: (0,) * len(s))

    out, loss, perp = pl.pallas_call(
        _k_net,
        grid=(bsz // bb,),
        in_specs=[
            pl.BlockSpec((bb, 8, 8, 48), lambda i: (i, 0, 0, 0)),
            const(4, 4, 3, 32), const(4, 4, 32, 64),
            const(5, 9 * _DIM, _DIM), const(4, _DIM, _DIM),
            const(_K, _DIM),
            const(4, 4, 64, 32), const(4, 4, 32, 3),
            const(1, 32), const(1, 32), const(1, 32),
            const(1, 64), const(1, 64), const(1, 64), const(1, 64),
            const(4, 64), const(4, 64), const(4, 64),
            const(4, 64), const(4, 64), const(4, 64),
            const(1, 32), const(1, 32), const(1, 32), const(1, 3),
        ],
        out_specs=[
            pl.BlockSpec((bb, 8, 8, 48), lambda i: (i, 0, 0, 0)),
            pl.BlockSpec((1, 1), lambda i: (0, 0)),
            pl.BlockSpec((1, 1), lambda i: (0, 0)),
        ],
        out_shape=[
            jax.ShapeDtypeStruct((bsz, 8, 8, 48), jnp.float32),
            jax.ShapeDtypeStruct((1, 1), jnp.float32),
            jax.ShapeDtypeStruct((1, 1), jnp.float32),
        ],
        scratch_shapes=[
            pltpu.VMEM((bb, 10, 10, 48), jnp.float32),
            pltpu.VMEM((bb, 10, 10, 128), jnp.float32),
            pltpu.VMEM((bb, 10, 10, 64), jnp.float32),
            pltpu.VMEM((9 * 48, 128), jnp.float32),
            pltpu.VMEM((9 * 128, 64), jnp.float32),
            pltpu.VMEM((9 * 64, 128), jnp.float32),
            pltpu.VMEM((9 * 128, 48), jnp.float32),
            pltpu.VMEM((1, 1), jnp.float32),
            pltpu.VMEM((1, _K), jnp.float32),
        ],
        compiler_params=pltpu.CompilerParams(
            dimension_semantics=("arbitrary",)),
    )(xs, w1t, w2t, w33, wrb2, p['codebook'], d1t, d2t,
      row(p['e_c1_b']), row(p['e_bn1_g']), row(p['e_bn1_b']),
      row(p['e_c2_b']), row(p['e_bn2_g']), row(p['e_bn2_b']),
      row(p['e_c3_b']),
      st4('_b1'), st4('_g1'), st4('_be1'),
      st4('_b2'), st4('_g2'), st4('_be2'),
      row(p['d_c1_b']), row(p['d_bn1_g']), row(p['d_bn1_b']),
      row(p['d_c2_b']))

    r = out.reshape(bsz, 8, 8, 4, 4, 3).transpose(0, 1, 3, 2, 4, 5)
    r = r.reshape(bsz, 32, 32, 3).transpose(0, 3, 1, 2)
    return r, loss[0, 0], perp[0, 0]


def kernel(x, params):
    return _run(x, params)
